# baseline (speedup 1.0000x reference)
import jax
import jax.numpy as jnp
from jax import lax
from jax.experimental import pallas as pl
from jax.experimental.pallas import tpu as pltpu
from jax.experimental.pallas import tpu_sc as plsc

N = 10000
E = 320000
D = 128
DW = D // 2

NC = 2
NS = 16
C = 128
NB = 3
CH0 = 102
CH1 = 57
CH_MAX = max(CH0, CH1)
TOT_CH = NS * (CH0 + CH1)
E_PAD = TOT_CH * C

LANES = 16
N_PAD = 10240
ROWS_PER_SUB = N_PAD // NS
FL = ROWS_PER_SUB // C


def _sc_segment_sum(ego_hbm, src_hbm, dst_hbm, wb_hbm, out_hbm,
                    src_v, dst_c, wb_v, rin_v, rbf_v, acc, gsem, ssem):
    cid = lax.axis_index("c")
    sid = lax.axis_index("s")
    my_ch = jnp.where(cid == 0, CH0, CH1)
    start0 = jnp.where(cid == 0, sid * CH0, NS * CH0 + sid * CH1)

    def src_slice(c):
        return src_v.at[pl.ds(pl.multiple_of(c * C, C), C)]

    def start_chunk(c, b):
        gi = start0 + c
        pltpu.async_copy(ego_hbm.at[src_slice(c)], rin_v[b], gsem[b])
        pltpu.async_copy(wb_hbm.at[gi], wb_v[b], gsem[b])
        pltpu.async_copy(dst_hbm.at[gi], dst_c.at[b], gsem[b])

    def wait_chunk(c, b):
        gi = start0 + c
        pltpu.make_async_copy(ego_hbm.at[src_slice(c)], rin_v[b], gsem[b]).wait()
        pltpu.make_async_copy(wb_hbm.at[gi], wb_v[b], gsem[b]).wait()
        pltpu.make_async_copy(dst_hbm.at[gi], dst_c.at[b], gsem[b]).wait()

    def start_scatter(b):
        pltpu.async_copy(rbf_v[b], acc.at[dst_c.at[b]], ssem[b], add=True)

    def wait_scatter(b):
        pltpu.make_async_copy(rbf_v[b], acc.at[dst_c.at[b]], ssem[b]).wait()

    zero32 = jnp.zeros((2 * LANES,), jnp.bfloat16)

    for i in range(C):
        for g in range(D // (2 * LANES)):
            rbf_v[0][i, pl.ds(g * 2 * LANES, 2 * LANES)] = zero32

    def zero_acc(k, _):
        pltpu.sync_copy(rbf_v[0],
                        acc.at[pl.ds(sid * ROWS_PER_SUB + k * C, C)])
        return 0
    lax.fori_loop(0, FL, zero_acc, 0)

    pltpu.sync_copy(
        src_hbm.at[pl.ds(pl.multiple_of(start0 * C, C), CH_MAX * C)], src_v)

    plsc.subcore_barrier()

    start_chunk(0, 0)

    def group_body(g, _):
        for b in range(NB):
            c = g * NB + b
            wait_chunk(c, b)
            nb = (b + 1) % NB

            @pl.when(c + 1 < my_ch)
            def _():
                @pl.when(c + 1 >= NB)
                def _():
                    wait_scatter(nb)
                start_chunk(c + 1, nb)

            for e in range(C):
                ws = wb_v[b][pl.ds(e * 2 * LANES, 2 * LANES)]
                for g2 in range(D // (2 * LANES)):
                    xb = rin_v[b][e, pl.ds(g2 * 2 * LANES, 2 * LANES)]
                    rbf_v[b][e, pl.ds(g2 * 2 * LANES, 2 * LANES)] = xb * ws

            start_scatter(b)
        return 0
    lax.fori_loop(0, my_ch // NB, group_body, 0)

    for b in range(NB):
        wait_scatter(b)

    plsc.subcore_barrier()

    pltpu.sync_copy(acc.at[pl.ds(sid * ROWS_PER_SUB, ROWS_PER_SUB)],
                    out_hbm.at[cid, pl.ds(sid * ROWS_PER_SUB, ROWS_PER_SUB)])


def _segment_sum_sc(ego, src, dst, w):
    ego_bf = ego.astype(jnp.bfloat16)

    pad = E_PAD - E
    src_p = jnp.concatenate([src, jnp.zeros((pad,), jnp.int32)])
    dst_p = jnp.concatenate([dst, jnp.zeros((pad,), jnp.int32)]).reshape(TOT_CH, C)
    w_p = jnp.concatenate([w, jnp.zeros((pad,), jnp.float32)]).astype(jnp.bfloat16)
    w_b = jnp.broadcast_to(w_p[:, None],
                           (E_PAD, 2 * LANES)).reshape(TOT_CH, C * 2 * LANES)

    mesh = plsc.VectorSubcoreMesh(core_axis_name="c", subcore_axis_name="s")
    call = pl.kernel(
        _sc_segment_sum,
        out_type=jax.ShapeDtypeStruct((NC, N_PAD, D), jnp.bfloat16),
        mesh=mesh,
        compiler_params=pltpu.CompilerParams(use_tc_tiling_on_sc=False),
        scratch_types=[
            pltpu.VMEM((CH_MAX * C,), jnp.int32),
            pltpu.VMEM((NB, C), jnp.int32),
            [pltpu.VMEM((C * 2 * LANES,), jnp.bfloat16) for _ in range(NB)],
            [pltpu.VMEM((C, D), jnp.bfloat16) for _ in range(NB)],
            [pltpu.VMEM((C, D), jnp.bfloat16) for _ in range(NB)],
            pltpu.VMEM_SHARED((N_PAD, D), jnp.bfloat16),
            [pltpu.SemaphoreType.DMA for _ in range(NB)],
            [pltpu.SemaphoreType.DMA for _ in range(NB)],
        ],
    )
    return call(ego_bf, src_p, dst_p, w_b)


def _dense_body(ego_ref, p0_ref, p1_ref, w_ref, b_ref, g_ref, bt_ref, out_ref):
    hi = (ego_ref[...] + p0_ref[...].astype(jnp.float32)
          + p1_ref[...].astype(jnp.float32))
    y = lax.dot_general(hi, w_ref[...], (((1,), (1,)), ((), ())),
                        preferred_element_type=jnp.float32)
    y = y + b_ref[...]
    y = jnp.where(y >= 0, y, 0.01 * y)
    mu = jnp.mean(y, axis=-1, keepdims=True)
    var = jnp.mean((y - mu) * (y - mu), axis=-1, keepdims=True)
    out_ref[...] = (y - mu) * lax.rsqrt(var + 1e-5) * g_ref[...] + bt_ref[...]


def _dense_tc(ego, p0, p1, W_lin, b_lin, ln_gamma, ln_beta):
    BR = 1000
    grid = (N // BR,)
    row_spec = pl.BlockSpec((BR, D), lambda i: (i, 0))
    full_spec = pl.BlockSpec((D, D), lambda i: (0, 0))
    vec_spec = pl.BlockSpec((1, D), lambda i: (0, 0))
    return pl.pallas_call(
        _dense_body,
        grid=grid,
        in_specs=[row_spec, row_spec, row_spec, full_spec, vec_spec, vec_spec,
                  vec_spec],
        out_specs=row_spec,
        out_shape=jax.ShapeDtypeStruct((N, D), jnp.float32),
    )(ego, p0, p1, W_lin, b_lin.reshape(1, D), ln_gamma.reshape(1, D),
      ln_beta.reshape(1, D))


@jax.jit
def kernel(ego_embeddings, edge_index, edge_weight, W_lin, b_lin, ln_gamma,
           ln_beta):
    src = edge_index[0]
    dst = edge_index[1]
    partials = _segment_sum_sc(ego_embeddings, src, dst, edge_weight)
    return _dense_tc(ego_embeddings, partials[0], partials[1], W_lin, b_lin,
                     ln_gamma, ln_beta)


# speedup vs baseline: 1.0178x; 1.0178x over previous
import jax
import jax.numpy as jnp
from jax import lax
from jax.experimental import pallas as pl
from jax.experimental.pallas import tpu as pltpu
from jax.experimental.pallas import tpu_sc as plsc

N = 10000
E = 320000
D = 128
DW = D // 2

NC = 2
NS = 16
C = 128
NB = 3
CH0 = 108
CH1 = 51
CH_MAX = max(CH0, CH1)
TOT_CH = NS * (CH0 + CH1)
E_PAD = TOT_CH * C

LANES = 16
N_PAD = 10240
ROWS_PER_SUB = N_PAD // NS
FL = ROWS_PER_SUB // C


def _sc_segment_sum(ego_hbm, src_hbm, dst_hbm, wb_hbm, out_hbm,
                    src_v, dst_c, wb_v, rin_v, rbf_v, acc, gsem, ssem):
    cid = lax.axis_index("c")
    sid = lax.axis_index("s")
    my_ch = jnp.where(cid == 0, CH0, CH1)
    start0 = jnp.where(cid == 0, sid * CH0, NS * CH0 + sid * CH1)

    def src_slice(c):
        return src_v.at[pl.ds(pl.multiple_of(c * C, C), C)]

    def start_chunk(c, b):
        gi = start0 + c
        pltpu.async_copy(ego_hbm.at[src_slice(c)], rin_v[b], gsem[b])
        pltpu.async_copy(wb_hbm.at[gi], wb_v[b], gsem[b])
        pltpu.async_copy(dst_hbm.at[gi], dst_c.at[b], gsem[b])

    def wait_chunk(c, b):
        gi = start0 + c
        pltpu.make_async_copy(ego_hbm.at[src_slice(c)], rin_v[b], gsem[b]).wait()
        pltpu.make_async_copy(wb_hbm.at[gi], wb_v[b], gsem[b]).wait()
        pltpu.make_async_copy(dst_hbm.at[gi], dst_c.at[b], gsem[b]).wait()

    def start_scatter(b):
        pltpu.async_copy(rbf_v[b], acc.at[dst_c.at[b]], ssem[b], add=True)

    def wait_scatter(b):
        pltpu.make_async_copy(rbf_v[b], acc.at[dst_c.at[b]], ssem[b]).wait()

    zero32 = jnp.zeros((2 * LANES,), jnp.bfloat16)

    for i in range(C):
        for g in range(D // (2 * LANES)):
            rbf_v[0][i, pl.ds(g * 2 * LANES, 2 * LANES)] = zero32

    def zero_acc(k, _):
        pltpu.sync_copy(rbf_v[0],
                        acc.at[pl.ds(sid * ROWS_PER_SUB + k * C, C)])
        return 0
    lax.fori_loop(0, FL, zero_acc, 0)

    pltpu.sync_copy(
        src_hbm.at[pl.ds(pl.multiple_of(start0 * C, C), CH_MAX * C)], src_v)

    plsc.subcore_barrier()

    start_chunk(0, 0)

    def group_body(g, _):
        for b in range(NB):
            c = g * NB + b
            wait_chunk(c, b)
            nb = (b + 1) % NB

            @pl.when(c + 1 < my_ch)
            def _():
                @pl.when(c + 1 >= NB)
                def _():
                    wait_scatter(nb)
                start_chunk(c + 1, nb)

            for e in range(C):
                ws = wb_v[b][pl.ds(e * 2 * LANES, 2 * LANES)]
                for g2 in range(D // (2 * LANES)):
                    xb = rin_v[b][e, pl.ds(g2 * 2 * LANES, 2 * LANES)]
                    rbf_v[b][e, pl.ds(g2 * 2 * LANES, 2 * LANES)] = xb * ws

            start_scatter(b)
        return 0
    lax.fori_loop(0, my_ch // NB, group_body, 0)

    for b in range(NB):
        wait_scatter(b)

    plsc.subcore_barrier()

    pltpu.sync_copy(acc.at[pl.ds(sid * ROWS_PER_SUB, ROWS_PER_SUB)],
                    out_hbm.at[cid, pl.ds(sid * ROWS_PER_SUB, ROWS_PER_SUB)])


def _segment_sum_sc(ego, src, dst, w):
    ego_bf = ego.astype(jnp.bfloat16)

    pad = E_PAD - E
    src_p = jnp.concatenate([src, jnp.zeros((pad,), jnp.int32)])
    dst_p = jnp.concatenate([dst, jnp.zeros((pad,), jnp.int32)]).reshape(TOT_CH, C)
    w_p = jnp.concatenate([w, jnp.zeros((pad,), jnp.float32)]).astype(jnp.bfloat16)
    w_b = jnp.broadcast_to(w_p[:, None],
                           (E_PAD, 2 * LANES)).reshape(TOT_CH, C * 2 * LANES)

    mesh = plsc.VectorSubcoreMesh(core_axis_name="c", subcore_axis_name="s")
    call = pl.kernel(
        _sc_segment_sum,
        out_type=jax.ShapeDtypeStruct((NC, N_PAD, D), jnp.bfloat16),
        mesh=mesh,
        compiler_params=pltpu.CompilerParams(use_tc_tiling_on_sc=False),
        scratch_types=[
            pltpu.VMEM((CH_MAX * C,), jnp.int32),
            pltpu.VMEM((NB, C), jnp.int32),
            [pltpu.VMEM((C * 2 * LANES,), jnp.bfloat16) for _ in range(NB)],
            [pltpu.VMEM((C, D), jnp.bfloat16) for _ in range(NB)],
            [pltpu.VMEM((C, D), jnp.bfloat16) for _ in range(NB)],
            pltpu.VMEM_SHARED((N_PAD, D), jnp.bfloat16),
            [pltpu.SemaphoreType.DMA for _ in range(NB)],
            [pltpu.SemaphoreType.DMA for _ in range(NB)],
        ],
    )
    return call(ego_bf, src_p, dst_p, w_b)


def _dense_body(ego_ref, p0_ref, p1_ref, w_ref, b_ref, g_ref, bt_ref, out_ref):
    hi = (ego_ref[...] + p0_ref[...].astype(jnp.float32)
          + p1_ref[...].astype(jnp.float32))
    y = lax.dot_general(hi, w_ref[...], (((1,), (1,)), ((), ())),
                        preferred_element_type=jnp.float32)
    y = y + b_ref[...]
    y = jnp.where(y >= 0, y, 0.01 * y)
    mu = jnp.mean(y, axis=-1, keepdims=True)
    var = jnp.mean((y - mu) * (y - mu), axis=-1, keepdims=True)
    out_ref[...] = (y - mu) * lax.rsqrt(var + 1e-5) * g_ref[...] + bt_ref[...]


def _dense_tc(ego, p0, p1, W_lin, b_lin, ln_gamma, ln_beta):
    BR = 1000
    grid = (N // BR,)
    row_spec = pl.BlockSpec((BR, D), lambda i: (i, 0))
    full_spec = pl.BlockSpec((D, D), lambda i: (0, 0))
    vec_spec = pl.BlockSpec((1, D), lambda i: (0, 0))
    return pl.pallas_call(
        _dense_body,
        grid=grid,
        in_specs=[row_spec, row_spec, row_spec, full_spec, vec_spec, vec_spec,
                  vec_spec],
        out_specs=row_spec,
        out_shape=jax.ShapeDtypeStruct((N, D), jnp.float32),
    )(ego, p0, p1, W_lin, b_lin.reshape(1, D), ln_gamma.reshape(1, D),
      ln_beta.reshape(1, D))


@jax.jit
def kernel(ego_embeddings, edge_index, edge_weight, W_lin, b_lin, ln_gamma,
           ln_beta):
    src = edge_index[0]
    dst = edge_index[1]
    partials = _segment_sum_sc(ego_embeddings, src, dst, edge_weight)
    return _dense_tc(ego_embeddings, partials[0], partials[1], W_lin, b_lin,
                     ln_gamma, ln_beta)


# speedup vs baseline: 1.0283x; 1.0103x over previous
import jax
import jax.numpy as jnp
from jax import lax
from jax.experimental import pallas as pl
from jax.experimental.pallas import tpu as pltpu
from jax.experimental.pallas import tpu_sc as plsc

N = 10000
E = 320000
D = 128
DW = D // 2

NC = 2
NS = 16
C = 128
NB = 3
CH0 = 111
CH1 = 48
CH_MAX = max(CH0, CH1)
TOT_CH = NS * (CH0 + CH1)
E_PAD = TOT_CH * C

LANES = 16
N_PAD = 10240
ROWS_PER_SUB = N_PAD // NS
FL = ROWS_PER_SUB // C


def _sc_segment_sum(ego_hbm, src_hbm, dst_hbm, wb_hbm, out_hbm,
                    src_v, dst_c, wb_v, rin_v, rbf_v, acc, gsem, ssem):
    cid = lax.axis_index("c")
    sid = lax.axis_index("s")
    my_ch = jnp.where(cid == 0, CH0, CH1)
    start0 = jnp.where(cid == 0, sid * CH0, NS * CH0 + sid * CH1)

    def src_slice(c):
        return src_v.at[pl.ds(pl.multiple_of(c * C, C), C)]

    def start_chunk(c, b):
        gi = start0 + c
        pltpu.async_copy(ego_hbm.at[src_slice(c)], rin_v[b], gsem[b])
        pltpu.async_copy(wb_hbm.at[gi], wb_v[b], gsem[b])
        pltpu.async_copy(dst_hbm.at[gi], dst_c.at[b], gsem[b])

    def wait_chunk(c, b):
        gi = start0 + c
        pltpu.make_async_copy(ego_hbm.at[src_slice(c)], rin_v[b], gsem[b]).wait()
        pltpu.make_async_copy(wb_hbm.at[gi], wb_v[b], gsem[b]).wait()
        pltpu.make_async_copy(dst_hbm.at[gi], dst_c.at[b], gsem[b]).wait()

    def start_scatter(b):
        pltpu.async_copy(rbf_v[b], acc.at[dst_c.at[b]], ssem[b], add=True)

    def wait_scatter(b):
        pltpu.make_async_copy(rbf_v[b], acc.at[dst_c.at[b]], ssem[b]).wait()

    zero32 = jnp.zeros((2 * LANES,), jnp.bfloat16)

    for i in range(C):
        for g in range(D // (2 * LANES)):
            rbf_v[0][i, pl.ds(g * 2 * LANES, 2 * LANES)] = zero32

    def zero_acc(k, _):
        pltpu.sync_copy(rbf_v[0],
                        acc.at[pl.ds(sid * ROWS_PER_SUB + k * C, C)])
        return 0
    lax.fori_loop(0, FL, zero_acc, 0)

    pltpu.sync_copy(
        src_hbm.at[pl.ds(pl.multiple_of(start0 * C, C), CH_MAX * C)], src_v)

    plsc.subcore_barrier()

    start_chunk(0, 0)

    def group_body(g, _):
        for b in range(NB):
            c = g * NB + b
            wait_chunk(c, b)
            nb = (b + 1) % NB

            @pl.when(c + 1 < my_ch)
            def _():
                @pl.when(c + 1 >= NB)
                def _():
                    wait_scatter(nb)
                start_chunk(c + 1, nb)

            for e in range(C):
                ws = wb_v[b][pl.ds(e * 2 * LANES, 2 * LANES)]
                for g2 in range(D // (2 * LANES)):
                    xb = rin_v[b][e, pl.ds(g2 * 2 * LANES, 2 * LANES)]
                    rbf_v[b][e, pl.ds(g2 * 2 * LANES, 2 * LANES)] = xb * ws

            start_scatter(b)
        return 0
    lax.fori_loop(0, my_ch // NB, group_body, 0)

    for b in range(NB):
        wait_scatter(b)

    plsc.subcore_barrier()

    pltpu.sync_copy(acc.at[pl.ds(sid * ROWS_PER_SUB, ROWS_PER_SUB)],
                    out_hbm.at[cid, pl.ds(sid * ROWS_PER_SUB, ROWS_PER_SUB)])


def _segment_sum_sc(ego, src, dst, w):
    ego_bf = ego.astype(jnp.bfloat16)

    pad = E_PAD - E
    src_p = jnp.concatenate([src, jnp.zeros((pad,), jnp.int32)])
    dst_p = jnp.concatenate([dst, jnp.zeros((pad,), jnp.int32)]).reshape(TOT_CH, C)
    w_p = jnp.concatenate([w, jnp.zeros((pad,), jnp.float32)]).astype(jnp.bfloat16)
    w_b = jnp.broadcast_to(w_p[:, None],
                           (E_PAD, 2 * LANES)).reshape(TOT_CH, C * 2 * LANES)

    mesh = plsc.VectorSubcoreMesh(core_axis_name="c", subcore_axis_name="s")
    call = pl.kernel(
        _sc_segment_sum,
        out_type=jax.ShapeDtypeStruct((NC, N_PAD, D), jnp.bfloat16),
        mesh=mesh,
        compiler_params=pltpu.CompilerParams(use_tc_tiling_on_sc=False),
        scratch_types=[
            pltpu.VMEM((CH_MAX * C,), jnp.int32),
            pltpu.VMEM((NB, C), jnp.int32),
            [pltpu.VMEM((C * 2 * LANES,), jnp.bfloat16) for _ in range(NB)],
            [pltpu.VMEM((C, D), jnp.bfloat16) for _ in range(NB)],
            [pltpu.VMEM((C, D), jnp.bfloat16) for _ in range(NB)],
            pltpu.VMEM_SHARED((N_PAD, D), jnp.bfloat16),
            [pltpu.SemaphoreType.DMA for _ in range(NB)],
            [pltpu.SemaphoreType.DMA for _ in range(NB)],
        ],
    )
    return call(ego_bf, src_p, dst_p, w_b)


def _dense_body(ego_ref, p0_ref, p1_ref, w_ref, b_ref, g_ref, bt_ref, out_ref):
    hi = (ego_ref[...] + p0_ref[...].astype(jnp.float32)
          + p1_ref[...].astype(jnp.float32))
    y = lax.dot_general(hi, w_ref[...], (((1,), (1,)), ((), ())),
                        preferred_element_type=jnp.float32)
    y = y + b_ref[...]
    y = jnp.where(y >= 0, y, 0.01 * y)
    mu = jnp.mean(y, axis=-1, keepdims=True)
    var = jnp.mean((y - mu) * (y - mu), axis=-1, keepdims=True)
    out_ref[...] = (y - mu) * lax.rsqrt(var + 1e-5) * g_ref[...] + bt_ref[...]


def _dense_tc(ego, p0, p1, W_lin, b_lin, ln_gamma, ln_beta):
    BR = 1000
    grid = (N // BR,)
    row_spec = pl.BlockSpec((BR, D), lambda i: (i, 0))
    full_spec = pl.BlockSpec((D, D), lambda i: (0, 0))
    vec_spec = pl.BlockSpec((1, D), lambda i: (0, 0))
    return pl.pallas_call(
        _dense_body,
        grid=grid,
        in_specs=[row_spec, row_spec, row_spec, full_spec, vec_spec, vec_spec,
                  vec_spec],
        out_specs=row_spec,
        out_shape=jax.ShapeDtypeStruct((N, D), jnp.float32),
    )(ego, p0, p1, W_lin, b_lin.reshape(1, D), ln_gamma.reshape(1, D),
      ln_beta.reshape(1, D))


@jax.jit
def kernel(ego_embeddings, edge_index, edge_weight, W_lin, b_lin, ln_gamma,
           ln_beta):
    src = edge_index[0]
    dst = edge_index[1]
    partials = _segment_sum_sc(ego_embeddings, src, dst, edge_weight)
    return _dense_tc(ego_embeddings, partials[0], partials[1], W_lin, b_lin,
                     ln_gamma, ln_beta)
